# Initial kernel scaffold; baseline (speedup 1.0000x reference)
import functools

import jax
import jax.numpy as jnp
from jax import lax
from jax.experimental import pallas as pl
from jax.experimental.pallas import tpu as pltpu
from jax.experimental.pallas import tpu_sc as plsc

V, D, H, O = 1_000_000, 32, 128, 64
B, L = 4096, 200
CHUNK = 16
LP = 208
NCHUNKS = LP // CHUNK
NW = 32
BPW = B // NW


def _pool_body(xt_hbm, len_hbm, table_hbm, out_hbm,
               idx_v, len_v, out_v, rows0, rows1, sem0, sem1):
    wid = lax.axis_index("s") * 2 + lax.axis_index("c")
    base = wid * BPW
    pltpu.sync_copy(xt_hbm.at[pl.ds(base, BPW)], idx_v)
    pltpu.sync_copy(len_hbm.at[pl.ds(base, BPW)], len_v)

    def fire(b, rows, sem):
        ln = len_v[b]
        nc = lax.div(ln + (CHUNK - 1), CHUNK)

        def body(c, carry):
            idx = idx_v.at[b, pl.ds(c * CHUNK, CHUNK)]
            pltpu.async_copy(table_hbm.at[idx], rows.at[c], sem)
            return carry

        lax.fori_loop(0, nc, body, 0)

    def drain_acc_store(b, rows, sem):
        ln = len_v[b]
        nfull = lax.div(ln, CHUNK)
        rem = ln - nfull * CHUNK
        nc = nfull + jnp.where(rem > 0, 1, 0)

        def dbody(c, carry):
            pltpu.make_async_copy(
                table_hbm.at[idx_v.at[b, pl.ds(0, CHUNK)]], rows.at[0], sem
            ).wait()
            return carry

        lax.fori_loop(0, nc, dbody, 0)

        zero = jnp.zeros((16,), jnp.float32)

        def abody(c, carry):
            a0, a1 = carry
            for t in range(CHUNK):
                a0 = a0 + rows[c, t, 0:16]
                a1 = a1 + rows[c, t, 16:32]
            return a0, a1

        a0, a1 = lax.fori_loop(0, nfull, abody, (zero, zero))

        for t in range(CHUNK):
            keep = t < rem
            r0 = jnp.where(keep, rows[nfull, t, 0:16], zero)
            r1 = jnp.where(keep, rows[nfull, t, 16:32], zero)
            a0 = a0 + r0
            a1 = a1 + r1

        linv = 1.0 / jnp.maximum(ln.astype(jnp.float32), 1.0)
        out_v[b, 0:16] = a0 * linv
        out_v[b, 16:32] = a1 * linv

    fire(0, rows0, sem0)

    def outer(i, carry):
        b0 = i * 2
        b1 = b0 + 1
        fire(b1, rows1, sem1)
        drain_acc_store(b0, rows0, sem0)

        @pl.when(b1 + 1 < BPW)
        def _():
            fire(b1 + 1, rows0, sem0)

        drain_acc_store(b1, rows1, sem1)
        return carry

    lax.fori_loop(0, BPW // 2, outer, 0)
    pltpu.sync_copy(out_v, out_hbm.at[pl.ds(base, BPW)])


_pool = functools.partial(
    pl.kernel,
    out_type=jax.ShapeDtypeStruct((B, D), jnp.float32),
    mesh=plsc.VectorSubcoreMesh(
        core_axis_name="c", subcore_axis_name="s", num_cores=2, num_subcores=16
    ),
    scratch_types=[
        pltpu.VMEM((BPW, LP), jnp.int32),
        pltpu.VMEM((BPW,), jnp.int32),
        pltpu.VMEM((BPW, D), jnp.float32),
        pltpu.VMEM((NCHUNKS, CHUNK, D), jnp.float32),
        pltpu.VMEM((NCHUNKS, CHUNK, D), jnp.float32),
        pltpu.SemaphoreType.DMA,
        pltpu.SemaphoreType.DMA,
    ],
)(_pool_body)


def _ffnn_body(vec_ref, w1_ref, b1_ref, w2_ref, b2_ref, out_ref):
    x = vec_ref[...]
    h = jnp.maximum(
        jnp.dot(x, w1_ref[...], preferred_element_type=jnp.float32) + b1_ref[...],
        0.0,
    )
    lg = jnp.dot(h, w2_ref[...], preferred_element_type=jnp.float32) + b2_ref[...]
    m = jnp.max(lg, axis=1, keepdims=True)
    ex = jnp.exp(lg - m)
    out_ref[...] = lg - m - jnp.log(jnp.sum(ex, axis=1, keepdims=True))


def _ffnn(vec, W1, b1, W2, b2):
    RB = 512
    return pl.pallas_call(
        _ffnn_body,
        grid=(B // RB,),
        in_specs=[
            pl.BlockSpec((RB, D), lambda i: (i, 0)),
            pl.BlockSpec((D, H), lambda i: (0, 0)),
            pl.BlockSpec((1, H), lambda i: (0, 0)),
            pl.BlockSpec((H, O), lambda i: (0, 0)),
            pl.BlockSpec((1, O), lambda i: (0, 0)),
        ],
        out_specs=pl.BlockSpec((RB, O), lambda i: (i, 0)),
        out_shape=jax.ShapeDtypeStruct((B, O), jnp.float32),
    )(vec, W1, b1.reshape(1, H), W2, b2.reshape(1, O))


def kernel(input, lengths, table, W1, b1, W2, b2):
    xt = jnp.pad(input.T, ((0, 0), (0, LP - L)))
    vec = _pool(xt, lengths, table)
    return _ffnn(vec, W1, b1, W2, b2)


# speedup vs baseline: 2.2761x; 2.2761x over previous
import functools

import jax
import jax.numpy as jnp
from jax import lax
from jax.experimental import pallas as pl
from jax.experimental.pallas import tpu as pltpu
from jax.experimental.pallas import tpu_sc as plsc

V, D, H, O = 1_000_000, 32, 128, 64
B, L = 4096, 200
CHUNK = 16
LP = 208
NCHUNKS = LP // CHUNK
NW = 32
BPW = B // NW


def _pool_body(xt_hbm, len_hbm, table_hbm, out_hbm,
               idx_v, len_v, out_v, rows0, rows1, sem0, sem1):
    wid = lax.axis_index("s") * 2 + lax.axis_index("c")
    base = wid * BPW
    pltpu.sync_copy(xt_hbm.at[pl.ds(base, BPW)], idx_v)
    pltpu.sync_copy(len_hbm.at[pl.ds(base, BPW)], len_v)

    def fire(b, ln, rows, sem):
        nc = lax.div(ln + (CHUNK - 1), CHUNK)

        def body(c, carry):
            idx = idx_v.at[b, pl.ds(c * CHUNK, CHUNK)]
            pltpu.async_copy(table_hbm.at[idx], rows.at[c], sem)
            return carry

        lax.fori_loop(0, nc, body, 0)

    def drain_acc_store(b, ln, rows, sem):
        nfull = lax.div(ln, CHUNK)
        rem = ln - nfull * CHUNK
        nc = nfull + jnp.where(rem > 0, 1, 0)

        def dbody(c, carry):
            pltpu.make_async_copy(
                table_hbm.at[idx_v.at[b, pl.ds(0, CHUNK)]], rows.at[0], sem
            ).wait()
            return carry

        lax.fori_loop(0, nc, dbody, 0)

        zero = jnp.zeros((16,), jnp.float32)

        def abody(c, carry):
            a0, a1 = carry
            for t in range(CHUNK):
                a0 = a0 + rows[c, t, 0:16]
                a1 = a1 + rows[c, t, 16:32]
            return a0, a1

        a0, a1 = lax.fori_loop(0, nfull, abody, (zero, zero))

        for t in range(CHUNK):
            keep = t < rem
            r0 = jnp.where(keep, rows[nfull, t, 0:16], zero)
            r1 = jnp.where(keep, rows[nfull, t, 16:32], zero)
            a0 = a0 + r0
            a1 = a1 + r1

        lnv = jnp.full((16,), jnp.maximum(ln.astype(jnp.float32), 1.0))
        out_v[b, 0:16] = a0 / lnv
        out_v[b, 16:32] = a1 / lnv

    G = 16
    NG = BPW // G
    bufs = ((rows0, sem0), (rows1, sem1))

    lvec0 = len_v[pl.ds(0, G)]
    fire(0, lvec0[0], rows0, sem0)

    def outer(g, carry):
        g16 = g * G
        lvec = len_v[pl.ds(g16, G)]
        lvec_next = len_v[pl.ds(jnp.minimum(g16 + G, BPW - G), G)]
        for j in range(G):
            b = g16 + j
            rows_c, sem_c = bufs[j % 2]
            rows_n, sem_n = bufs[(j + 1) % 2]
            if j < G - 1:
                fire(b + 1, lvec[j + 1], rows_n, sem_n)
            else:
                @pl.when(g < NG - 1)
                def _():
                    fire(b + 1, lvec_next[0], rows_n, sem_n)
            drain_acc_store(b, lvec[j], rows_c, sem_c)
        return carry

    lax.fori_loop(0, NG, outer, 0)
    pltpu.sync_copy(out_v, out_hbm.at[pl.ds(base, BPW)])


_pool = functools.partial(
    pl.kernel,
    out_type=jax.ShapeDtypeStruct((B, D), jnp.float32),
    mesh=plsc.VectorSubcoreMesh(
        core_axis_name="c", subcore_axis_name="s", num_cores=2, num_subcores=16
    ),
    scratch_types=[
        pltpu.VMEM((BPW, LP), jnp.int32),
        pltpu.VMEM((BPW,), jnp.int32),
        pltpu.VMEM((BPW, D), jnp.float32),
        pltpu.VMEM((NCHUNKS, CHUNK, D), jnp.float32),
        pltpu.VMEM((NCHUNKS, CHUNK, D), jnp.float32),
        pltpu.SemaphoreType.DMA,
        pltpu.SemaphoreType.DMA,
    ],
    compiler_params=pltpu.CompilerParams(use_tc_tiling_on_sc=False),
)(_pool_body)


def _ffnn_body(vec_ref, w1_ref, b1_ref, w2_ref, b2_ref, out_ref):
    x = vec_ref[...]
    h = jnp.maximum(
        jnp.dot(x, w1_ref[...], preferred_element_type=jnp.float32) + b1_ref[...],
        0.0,
    )
    lg = jnp.dot(h, w2_ref[...], preferred_element_type=jnp.float32) + b2_ref[...]
    m = jnp.max(lg, axis=1, keepdims=True)
    ex = jnp.exp(lg - m)
    out_ref[...] = lg - m - jnp.log(jnp.sum(ex, axis=1, keepdims=True))


def _ffnn(vec, W1, b1, W2, b2):
    RB = 512
    return pl.pallas_call(
        _ffnn_body,
        grid=(B // RB,),
        in_specs=[
            pl.BlockSpec((RB, D), lambda i: (i, 0)),
            pl.BlockSpec((D, H), lambda i: (0, 0)),
            pl.BlockSpec((1, H), lambda i: (0, 0)),
            pl.BlockSpec((H, O), lambda i: (0, 0)),
            pl.BlockSpec((1, O), lambda i: (0, 0)),
        ],
        out_specs=pl.BlockSpec((RB, O), lambda i: (i, 0)),
        out_shape=jax.ShapeDtypeStruct((B, O), jnp.float32),
    )(vec, W1, b1.reshape(1, H), W2, b2.reshape(1, O))


def kernel(input, lengths, table, W1, b1, W2, b2):
    xt = jnp.pad(input.T, ((0, 0), (0, LP - L)))
    vec = _pool(xt, lengths, table)
    return _ffnn(vec, W1, b1, W2, b2)
